# baseline (speedup 1.0000x reference)
import jax
import jax.numpy as jnp
from jax import lax
from jax.experimental import pallas as pl
from jax.experimental.pallas import tpu as pltpu
from jax.experimental.pallas import tpu_sc as plsc

N_NODES = 10000
N_EDGES = 320000
D = 128
D_EDGE = 16
NUM_STEPS = 6

NC = 2
NS = 16
NW = NC * NS
EPW = N_EDGES // NW
CHUNK = 80
NCHUNKS = EPW // CHUNK
NPAD = 10112
RPT = NPAD // NS

_f32 = jnp.float32

_sc_mesh = plsc.VectorSubcoreMesh(
    core_axis_name="c", subcore_axis_name="s", num_cores=NC, num_subcores=NS)


SUPER = 400
GSUBS = ((0, 128), (128, 128), (256, 128), (384, 16))
NSUP = EPW // SUPER
NPAIR = (NSUP - 5) // 2
SNSUP = EPW // CHUNK
SNPAIR = (SNSUP - 5) // 2


def _gather_body(table, idx, out, idx_v, rows_v, si0, si1, sg, so0, so1):
    c = lax.axis_index("c")
    s = lax.axis_index("s")
    base = (s * NC + c) * EPW
    si = (si0, si1)
    so = (so0, so1)

    def fire_idx(g, b):
        for j, (off, ln) in enumerate(GSUBS):
            pltpu.async_copy(idx.at[pl.ds(base + g * SUPER + off, ln)],
                             idx_v.at[b, j, pl.ds(0, ln)], si[b])

    def drain_idx(b):
        for j, (off, ln) in enumerate(GSUBS):
            pltpu.make_async_copy(idx.at[pl.ds(base, ln)],
                                  idx_v.at[b, j, pl.ds(0, ln)], si[b]).wait()

    def drain_store(b):
        pltpu.make_async_copy(out.at[pl.ds(base, SUPER)],
                              rows_v.at[b], so[b]).wait()

    def step(g, b, first=False, fire_next=True):
        if not first:
            drain_store(b)
        drain_idx(b)
        cps = [pltpu.async_copy(table.at[idx_v.at[b, j, pl.ds(0, ln)]],
                                rows_v.at[b, pl.ds(off, ln)], sg)
               for j, (off, ln) in enumerate(GSUBS)]
        for cp in cps:
            cp.wait()
        if fire_next:
            fire_idx(g + 2, b)
        pltpu.async_copy(rows_v.at[b], out.at[pl.ds(base + g * SUPER, SUPER)],
                         so[b])

    fire_idx(0, 0)
    fire_idx(1, 1)
    step(0, 0, first=True)
    step(1, 1, first=True)

    def body(p, carry):
        g = 2 + 2 * p
        step(g, 0)
        step(g + 1, 1)
        return carry

    lax.fori_loop(0, NPAIR, body, 0)
    step(NSUP - 3, 0)
    step(NSUP - 2, 1, fire_next=False)
    step(NSUP - 1, 0, fire_next=False)
    drain_store(1)
    drain_store(0)


def _sc_gather(table, idx, dtype=_f32, width=D):
    return pl.kernel(
        _gather_body,
        out_type=jax.ShapeDtypeStruct((N_EDGES, width), dtype),
        mesh=_sc_mesh,
        scratch_types=[
            pltpu.VMEM((2, len(GSUBS), 128), jnp.int32),
            pltpu.VMEM((2, SUPER, width), dtype),
            pltpu.SemaphoreType.DMA,
            pltpu.SemaphoreType.DMA,
            pltpu.SemaphoreType.DMA,
            pltpu.SemaphoreType.DMA,
            pltpu.SemaphoreType.DMA,
        ],
    )(table, idx)


SNB = 4


def _scatter_body(vals, idx, zeros, out, idx_v, rows_v, acc,
                  sl0, sl1, sl2, sl3, sc0, sc1, sc2, sc3):
    c = lax.axis_index("c")
    s = lax.axis_index("s")
    base = (s * NC + c) * EPW
    sl = (sl0, sl1, sl2, sl3)
    ssc = (sc0, sc1, sc2, sc3)

    pltpu.sync_copy(zeros.at[pl.ds(s * RPT, RPT)], acc.at[pl.ds(s * RPT, RPT)])

    def fire_load(g, b):
        pltpu.async_copy(idx.at[pl.ds(base + g * CHUNK, CHUNK)],
                         idx_v.at[b], sl[b])
        pltpu.async_copy(vals.at[pl.ds(base + g * CHUNK, CHUNK)],
                         rows_v.at[b], sl[b])

    def drain_load(b):
        pltpu.make_async_copy(idx.at[pl.ds(base, CHUNK)],
                              idx_v.at[b], sl[b]).wait()
        pltpu.make_async_copy(vals.at[pl.ds(base, CHUNK)],
                              rows_v.at[b], sl[b]).wait()

    def drain_add(b):
        pltpu.make_async_copy(vals.at[pl.ds(base, CHUNK)],
                              rows_v.at[b], ssc[b]).wait()

    fire_load(0, 0)
    fire_load(1, 1)
    plsc.subcore_barrier()

    def step(g, b, drain_prev=True, fire_next=True):
        drain_load(b)
        pltpu.async_copy(rows_v.at[b], acc.at[idx_v.at[b]], ssc[b], add=True)
        bn = (b + 2) % SNB
        if drain_prev:
            drain_add(bn)
        if fire_next:
            fire_load(g + 2, bn)

    step(0, 0, drain_prev=False)
    step(1, 1, drain_prev=False)

    def body(p, carry):
        g = 2 + 4 * p
        for j in range(4):
            step(g + j, (2 + j) % SNB)
        return carry

    lax.fori_loop(0, (SNSUP - 5) // 4, body, 0)
    step(SNSUP - 3, (SNSUP - 3) % SNB)
    step(SNSUP - 2, (SNSUP - 2) % SNB, fire_next=False)
    step(SNSUP - 1, (SNSUP - 1) % SNB, fire_next=False)
    drain_add((SNSUP - 2) % SNB)
    drain_add((SNSUP - 1) % SNB)

    plsc.subcore_barrier()
    pltpu.sync_copy(acc.at[pl.ds(s * RPT, RPT)],
                    out.at[c, pl.ds(s * RPT, RPT)])


def _sc_scatter(vals, idx, zeros):
    return pl.kernel(
        _scatter_body,
        out_type=jax.ShapeDtypeStruct((NC, NPAD, D), _f32),
        mesh=_sc_mesh,
        scratch_types=[
            pltpu.VMEM((SNB, CHUNK), jnp.int32),
            pltpu.VMEM((SNB, CHUNK, D), _f32),
            pltpu.VMEM_SHARED((NPAD, D), _f32),
        ] + [pltpu.SemaphoreType.DMA] * (2 * SNB),
    )(vals, idx, zeros)


EPT = N_EDGES // NS
FSN = EPT // CHUNK
FK = (FSN - 6) // 4
GN = EPW // CHUNK
GPAIR = (GN - 5) // 2


def _scatgath_body(vals, dsti, srci, zeros, out, idx_v, rows_v, acc,
                   sl0, sl1, sl2, sl3, sc0, sc1, sc2, sc3):
    c = lax.axis_index("c")
    s = lax.axis_index("s")
    sl = (sl0, sl1, sl2, sl3)
    ssc = (sc0, sc1, sc2, sc3)

    sbase = s * EPT
    pltpu.sync_copy(zeros.at[pl.ds(s * RPT, RPT)], acc.at[pl.ds(s * RPT, RPT)])

    def fire_load(g, b):
        pltpu.async_copy(dsti.at[pl.ds(sbase + g * CHUNK, CHUNK)],
                         idx_v.at[b], sl[b])
        pltpu.async_copy(vals.at[pl.ds(sbase + g * CHUNK, CHUNK)],
                         rows_v.at[b], sl[b])

    def drain_load(b):
        pltpu.make_async_copy(dsti.at[pl.ds(sbase, CHUNK)],
                              idx_v.at[b], sl[b]).wait()
        pltpu.make_async_copy(vals.at[pl.ds(sbase, CHUNK)],
                              rows_v.at[b], sl[b]).wait()

    def drain_add(b):
        pltpu.make_async_copy(vals.at[pl.ds(sbase, CHUNK)],
                              rows_v.at[b], ssc[b]).wait()

    fire_load(0, 0)
    fire_load(1, 1)
    plsc.subcore_barrier()

    def sstep(g, b, drain_prev=True, fire_next=True):
        drain_load(b)
        pltpu.async_copy(rows_v.at[b], acc.at[idx_v.at[b]], ssc[b], add=True)
        bn = (b + 2) % SNB
        if drain_prev:
            drain_add(bn)
        if fire_next:
            fire_load(g + 2, bn)

    sstep(0, 0, drain_prev=False)
    sstep(1, 1, drain_prev=False)

    def sbody(p, carry):
        g = 2 + 4 * p
        for j in range(4):
            sstep(g + j, (2 + j) % SNB)
        return carry

    lax.fori_loop(0, FK, sbody, 0)
    for g in range(2 + 4 * FK, FSN):
        sstep(g, g % SNB, fire_next=g + 2 < FSN)
    drain_add((FSN - 2) % SNB)
    drain_add((FSN - 1) % SNB)

    plsc.subcore_barrier()

    gbase = (s * NC + c) * EPW
    si = (sl0, sl1)
    sg = sc0
    so = (sc1, sc2)

    def fire_idx(g, b):
        pltpu.async_copy(srci.at[pl.ds(gbase + g * CHUNK, CHUNK)],
                         idx_v.at[b], si[b])

    def drain_idx(b):
        pltpu.make_async_copy(srci.at[pl.ds(gbase, CHUNK)],
                              idx_v.at[b], si[b]).wait()

    def drain_store(b):
        pltpu.make_async_copy(out.at[pl.ds(gbase, CHUNK)],
                              rows_v.at[b], so[b]).wait()

    def gstep(g, b, first=False, fire_next=True):
        if not first:
            drain_store(b)
        drain_idx(b)
        pltpu.async_copy(acc.at[idx_v.at[b]], rows_v.at[b], sg).wait()
        if fire_next:
            fire_idx(g + 2, b)
        pltpu.async_copy(rows_v.at[b], out.at[pl.ds(gbase + g * CHUNK, CHUNK)],
                         so[b])

    fire_idx(0, 0)
    fire_idx(1, 1)
    gstep(0, 0, first=True)
    gstep(1, 1, first=True)

    def gbody(p, carry):
        g = 2 + 2 * p
        gstep(g, 0)
        gstep(g + 1, 1)
        return carry

    lax.fori_loop(0, GPAIR, gbody, 0)
    gstep(GN - 3, 0)
    gstep(GN - 2, 1, fire_next=False)
    gstep(GN - 1, 0, fire_next=False)
    drain_store(1)
    drain_store(0)


def _sc_scatgath(vals, dsti, srci, zeros):
    return pl.kernel(
        _scatgath_body,
        out_type=jax.ShapeDtypeStruct((N_EDGES, D), _f32),
        mesh=_sc_mesh,
        scratch_types=[
            pltpu.VMEM((SNB, CHUNK), jnp.int32),
            pltpu.VMEM((SNB, CHUNK, D), _f32),
            pltpu.VMEM_SHARED((NPAD, D), _f32),
        ] + [pltpu.SemaphoreType.DMA] * (2 * SNB),
    )(vals, dsti, srci, zeros)


BE = 4000
NBLK = N_EDGES // BE


def _node_proj_body(nf_ref, w_ref, o_ref):
    o_ref[...] = jnp.dot(nf_ref[...], w_ref[...],
                         preferred_element_type=_f32)


def _tc_node_proj(nf, w):
    return pl.pallas_call(
        _node_proj_body,
        out_shape=jax.ShapeDtypeStruct((N_NODES, D), _f32),
    )(nf, w)


def _init_body(ps_ref, ef_ref, wie_ref, wm_ref, h0_ref, q_ref):
    h0 = jnp.maximum(
        ps_ref[...] + jnp.dot(ef_ref[...], wie_ref[...],
                              preferred_element_type=_f32), 0.0)
    h0_ref[...] = h0.astype(jnp.bfloat16)
    q_ref[...] = jnp.dot(h0, wm_ref[...], preferred_element_type=_f32)


def _tc_init(psrc, ef, w_ie, w_m):
    return pl.pallas_call(
        _init_body,
        grid=(NBLK,),
        in_specs=[
            pl.BlockSpec((BE, D), lambda i: (i, 0)),
            pl.BlockSpec((BE, D_EDGE), lambda i: (i, 0)),
            pl.BlockSpec((D_EDGE, D), lambda i: (0, 0)),
            pl.BlockSpec((D, D), lambda i: (0, 0)),
        ],
        out_specs=[pl.BlockSpec((BE, D), lambda i: (i, 0))] * 2,
        out_shape=[jax.ShapeDtypeStruct((N_EDGES, D), jnp.bfloat16),
                   jax.ShapeDtypeStruct((N_EDGES, D), _f32)],
    )(psrc, ef, w_ie, w_m)


def _add_body(p_ref, o_ref):
    o_ref[...] = p_ref[0] + p_ref[1]


def _tc_add(parts):
    return pl.pallas_call(
        _add_body,
        out_shape=jax.ShapeDtypeStruct((NPAD, D), _f32),
    )(parts)


def _step_body(h0_ref, q_ref, g_ref, wm_ref, qn_ref):
    ef = jnp.maximum(h0_ref[...].astype(_f32) + g_ref[...]
                     - q_ref[...], 0.0)
    qn_ref[...] = jnp.dot(ef, wm_ref[...], preferred_element_type=_f32)


def _tc_step(h0, q, gsrc, w_m):
    return pl.pallas_call(
        _step_body,
        grid=(NBLK,),
        in_specs=[
            pl.BlockSpec((BE, D), lambda i: (i, 0)),
            pl.BlockSpec((BE, D), lambda i: (i, 0)),
            pl.BlockSpec((BE, D), lambda i: (i, 0)),
            pl.BlockSpec((D, D), lambda i: (0, 0)),
        ],
        out_specs=pl.BlockSpec((BE, D), lambda i: (i, 0)),
        out_shape=jax.ShapeDtypeStruct((N_EDGES, D), _f32),
    )(h0, q, gsrc, w_m)


def _ef_body(h0_ref, q_ref, g_ref, o_ref):
    o_ref[...] = jnp.maximum(h0_ref[...].astype(_f32) + g_ref[...]
                             - q_ref[...], 0.0)


def _tc_ef(h0, q, gsrc):
    return pl.pallas_call(
        _ef_body,
        grid=(NBLK,),
        in_specs=[
            pl.BlockSpec((BE, D), lambda i: (i, 0)),
            pl.BlockSpec((BE, D), lambda i: (i, 0)),
            pl.BlockSpec((BE, D), lambda i: (i, 0)),
        ],
        out_specs=pl.BlockSpec((BE, D), lambda i: (i, 0)),
        out_shape=jax.ShapeDtypeStruct((N_EDGES, D), _f32),
    )(h0, q, gsrc)


def _final_body(nf_ref, parts_ref, wan_ref, wao_ref, o_ref):
    onode = parts_ref[0, :N_NODES] + parts_ref[1, :N_NODES]
    o_ref[...] = jnp.maximum(
        jnp.dot(nf_ref[...], wan_ref[...], preferred_element_type=_f32)
        + jnp.dot(onode, wao_ref[...], preferred_element_type=_f32), 0.0)


def _tc_final(nf, parts, w_an, w_ao):
    return pl.pallas_call(
        _final_body,
        out_shape=jax.ShapeDtypeStruct((N_NODES, D), _f32),
    )(nf, parts, w_an, w_ao)


def kernel(node_feats, edge_feats, edge_index, W_i, W_m, W_a):
    src = edge_index[0].astype(jnp.int32)
    dst = edge_index[1].astype(jnp.int32)
    zeros = jnp.zeros((NPAD, D), _f32)

    p = _tc_node_proj(node_feats, W_i[:D])
    psrc = _sc_gather(p, src)
    h0, q = _tc_init(psrc, edge_feats, W_i[D:], W_m)

    for _ in range(NUM_STEPS - 1):
        gsrc = _sc_scatgath(q, dst, src, zeros)
        q = _tc_step(h0, q, gsrc, W_m)

    gsrc = _sc_scatgath(q, dst, src, zeros)
    ef6 = _tc_ef(h0, q, gsrc)

    parts = _sc_scatter(ef6, dst, zeros)
    return _tc_final(node_feats, parts, W_a[:D], W_a[D:])


# speedup vs baseline: 1.0564x; 1.0564x over previous
import jax
import jax.numpy as jnp
from jax import lax
from jax.experimental import pallas as pl
from jax.experimental.pallas import tpu as pltpu
from jax.experimental.pallas import tpu_sc as plsc

N_NODES = 10000
N_EDGES = 320000
D = 128
D_EDGE = 16
NUM_STEPS = 6

NC = 2
NS = 16
NW = NC * NS
EPW = N_EDGES // NW
CHUNK = 80
NCHUNKS = EPW // CHUNK
NPAD = 10112
RPT = NPAD // NS

_f32 = jnp.float32

_sc_mesh = plsc.VectorSubcoreMesh(
    core_axis_name="c", subcore_axis_name="s", num_cores=NC, num_subcores=NS)


SUPER = 400
GSUBS = ((0, 128), (128, 128), (256, 128), (384, 16))
NSUP = EPW // SUPER
NPAIR = (NSUP - 5) // 2
SNSUP = EPW // CHUNK
SNPAIR = (SNSUP - 5) // 2


def _gather_body(table, idx, out, idx_v, rows_v, si0, si1, sg, so0, so1):
    c = lax.axis_index("c")
    s = lax.axis_index("s")
    base = (s * NC + c) * EPW
    si = (si0, si1)
    so = (so0, so1)

    def fire_idx(g, b):
        for j, (off, ln) in enumerate(GSUBS):
            pltpu.async_copy(idx.at[pl.ds(base + g * SUPER + off, ln)],
                             idx_v.at[b, j, pl.ds(0, ln)], si[b])

    def drain_idx(b):
        for j, (off, ln) in enumerate(GSUBS):
            pltpu.make_async_copy(idx.at[pl.ds(base, ln)],
                                  idx_v.at[b, j, pl.ds(0, ln)], si[b]).wait()

    def drain_store(b):
        pltpu.make_async_copy(out.at[pl.ds(base, SUPER)],
                              rows_v.at[b], so[b]).wait()

    def step(g, b, first=False, fire_next=True):
        if not first:
            drain_store(b)
        drain_idx(b)
        cps = [pltpu.async_copy(table.at[idx_v.at[b, j, pl.ds(0, ln)]],
                                rows_v.at[b, pl.ds(off, ln)], sg)
               for j, (off, ln) in enumerate(GSUBS)]
        for cp in cps:
            cp.wait()
        if fire_next:
            fire_idx(g + 2, b)
        pltpu.async_copy(rows_v.at[b], out.at[pl.ds(base + g * SUPER, SUPER)],
                         so[b])

    fire_idx(0, 0)
    fire_idx(1, 1)
    step(0, 0, first=True)
    step(1, 1, first=True)

    def body(p, carry):
        g = 2 + 2 * p
        step(g, 0)
        step(g + 1, 1)
        return carry

    lax.fori_loop(0, NPAIR, body, 0)
    step(NSUP - 3, 0)
    step(NSUP - 2, 1, fire_next=False)
    step(NSUP - 1, 0, fire_next=False)
    drain_store(1)
    drain_store(0)


def _sc_gather(table, idx, dtype=_f32, width=D):
    return pl.kernel(
        _gather_body,
        out_type=jax.ShapeDtypeStruct((N_EDGES, width), dtype),
        mesh=_sc_mesh,
        scratch_types=[
            pltpu.VMEM((2, len(GSUBS), 128), jnp.int32),
            pltpu.VMEM((2, SUPER, width), dtype),
            pltpu.SemaphoreType.DMA,
            pltpu.SemaphoreType.DMA,
            pltpu.SemaphoreType.DMA,
            pltpu.SemaphoreType.DMA,
            pltpu.SemaphoreType.DMA,
        ],
    )(table, idx)


SNB = 4


def _scatter_body(vals, idx, zeros, out, idx_v, rows_v, acc,
                  sl0, sl1, sl2, sl3, sc0, sc1, sc2, sc3):
    c = lax.axis_index("c")
    s = lax.axis_index("s")
    base = (s * NC + c) * EPW
    sl = (sl0, sl1, sl2, sl3)
    ssc = (sc0, sc1, sc2, sc3)

    pltpu.sync_copy(zeros.at[pl.ds(s * RPT, RPT)], acc.at[pl.ds(s * RPT, RPT)])

    def fire_load(g, b):
        pltpu.async_copy(idx.at[pl.ds(base + g * CHUNK, CHUNK)],
                         idx_v.at[b], sl[b])
        pltpu.async_copy(vals.at[pl.ds(base + g * CHUNK, CHUNK)],
                         rows_v.at[b], sl[b])

    def drain_load(b):
        pltpu.make_async_copy(idx.at[pl.ds(base, CHUNK)],
                              idx_v.at[b], sl[b]).wait()
        pltpu.make_async_copy(vals.at[pl.ds(base, CHUNK)],
                              rows_v.at[b], sl[b]).wait()

    def drain_add(b):
        pltpu.make_async_copy(vals.at[pl.ds(base, CHUNK)],
                              rows_v.at[b], ssc[b]).wait()

    fire_load(0, 0)
    fire_load(1, 1)
    plsc.subcore_barrier()

    def step(g, b, drain_prev=True, fire_next=True):
        drain_load(b)
        pltpu.async_copy(rows_v.at[b], acc.at[idx_v.at[b]], ssc[b], add=True)
        bn = (b + 2) % SNB
        if drain_prev:
            drain_add(bn)
        if fire_next:
            fire_load(g + 2, bn)

    step(0, 0, drain_prev=False)
    step(1, 1, drain_prev=False)

    def body(p, carry):
        g = 2 + 4 * p
        for j in range(4):
            step(g + j, (2 + j) % SNB)
        return carry

    lax.fori_loop(0, (SNSUP - 5) // 4, body, 0)
    step(SNSUP - 3, (SNSUP - 3) % SNB)
    step(SNSUP - 2, (SNSUP - 2) % SNB, fire_next=False)
    step(SNSUP - 1, (SNSUP - 1) % SNB, fire_next=False)
    drain_add((SNSUP - 2) % SNB)
    drain_add((SNSUP - 1) % SNB)

    plsc.subcore_barrier()
    pltpu.sync_copy(acc.at[pl.ds(s * RPT, RPT)],
                    out.at[c, pl.ds(s * RPT, RPT)])


def _sc_scatter(vals, idx, zeros):
    return pl.kernel(
        _scatter_body,
        out_type=jax.ShapeDtypeStruct((NC, NPAD, D), _f32),
        mesh=_sc_mesh,
        scratch_types=[
            pltpu.VMEM((SNB, CHUNK), jnp.int32),
            pltpu.VMEM((SNB, CHUNK, D), _f32),
            pltpu.VMEM_SHARED((NPAD, D), _f32),
        ] + [pltpu.SemaphoreType.DMA] * (2 * SNB),
    )(vals, idx, zeros)


BE = 4000
NBLK = N_EDGES // BE


def _node_proj_body(nf_ref, w_ref, o_ref):
    o_ref[...] = jnp.dot(nf_ref[...], w_ref[...],
                         preferred_element_type=_f32)


def _tc_node_proj(nf, w):
    return pl.pallas_call(
        _node_proj_body,
        out_shape=jax.ShapeDtypeStruct((N_NODES, D), _f32),
    )(nf, w)


def _init_body(ps_ref, ef_ref, wie_ref, wm_ref, h0_ref, q_ref):
    h0 = jnp.maximum(
        ps_ref[...] + jnp.dot(ef_ref[...], wie_ref[...],
                              preferred_element_type=_f32), 0.0)
    h0_ref[...] = h0.astype(jnp.bfloat16)
    q_ref[...] = jnp.dot(h0, wm_ref[...], preferred_element_type=_f32)


def _tc_init(psrc, ef, w_ie, w_m):
    return pl.pallas_call(
        _init_body,
        grid=(NBLK,),
        in_specs=[
            pl.BlockSpec((BE, D), lambda i: (i, 0)),
            pl.BlockSpec((BE, D_EDGE), lambda i: (i, 0)),
            pl.BlockSpec((D_EDGE, D), lambda i: (0, 0)),
            pl.BlockSpec((D, D), lambda i: (0, 0)),
        ],
        out_specs=[pl.BlockSpec((BE, D), lambda i: (i, 0))] * 2,
        out_shape=[jax.ShapeDtypeStruct((N_EDGES, D), jnp.bfloat16),
                   jax.ShapeDtypeStruct((N_EDGES, D), _f32)],
    )(psrc, ef, w_ie, w_m)


def _add_body(p_ref, o_ref):
    o_ref[...] = p_ref[0] + p_ref[1]


def _tc_add(parts):
    return pl.pallas_call(
        _add_body,
        out_shape=jax.ShapeDtypeStruct((NPAD, D), _f32),
    )(parts)


def _step_body(h0_ref, q_ref, g_ref, wm_ref, qn_ref):
    ef = jnp.maximum(h0_ref[...].astype(_f32) + g_ref[...]
                     - q_ref[...], 0.0)
    qn_ref[...] = jnp.dot(ef, wm_ref[...], preferred_element_type=_f32)


def _tc_step(h0, q, gsrc, w_m):
    return pl.pallas_call(
        _step_body,
        grid=(NBLK,),
        in_specs=[
            pl.BlockSpec((BE, D), lambda i: (i, 0)),
            pl.BlockSpec((BE, D), lambda i: (i, 0)),
            pl.BlockSpec((BE, D), lambda i: (i, 0)),
            pl.BlockSpec((D, D), lambda i: (0, 0)),
        ],
        out_specs=pl.BlockSpec((BE, D), lambda i: (i, 0)),
        out_shape=jax.ShapeDtypeStruct((N_EDGES, D), _f32),
    )(h0, q, gsrc, w_m)


def _ef_body(h0_ref, q_ref, g_ref, o_ref):
    o_ref[...] = jnp.maximum(h0_ref[...].astype(_f32) + g_ref[...]
                             - q_ref[...], 0.0)


def _tc_ef(h0, q, gsrc):
    return pl.pallas_call(
        _ef_body,
        grid=(NBLK,),
        in_specs=[
            pl.BlockSpec((BE, D), lambda i: (i, 0)),
            pl.BlockSpec((BE, D), lambda i: (i, 0)),
            pl.BlockSpec((BE, D), lambda i: (i, 0)),
        ],
        out_specs=pl.BlockSpec((BE, D), lambda i: (i, 0)),
        out_shape=jax.ShapeDtypeStruct((N_EDGES, D), _f32),
    )(h0, q, gsrc)


def _final_body(nf_ref, parts_ref, wan_ref, wao_ref, o_ref):
    onode = parts_ref[0, :N_NODES] + parts_ref[1, :N_NODES]
    o_ref[...] = jnp.maximum(
        jnp.dot(nf_ref[...], wan_ref[...], preferred_element_type=_f32)
        + jnp.dot(onode, wao_ref[...], preferred_element_type=_f32), 0.0)


def _tc_final(nf, parts, w_an, w_ao):
    return pl.pallas_call(
        _final_body,
        out_shape=jax.ShapeDtypeStruct((N_NODES, D), _f32),
    )(nf, parts, w_an, w_ao)


def kernel(node_feats, edge_feats, edge_index, W_i, W_m, W_a):
    src = edge_index[0].astype(jnp.int32)
    dst = edge_index[1].astype(jnp.int32)
    zeros = jnp.zeros((NPAD, D), _f32)

    p = _tc_node_proj(node_feats, W_i[:D])
    psrc = _sc_gather(p, src)
    h0, q = _tc_init(psrc, edge_feats, W_i[D:], W_m)

    for _ in range(NUM_STEPS - 1):
        parts = _sc_scatter(q, dst, zeros)
        g = _tc_add(parts)
        gsrc = _sc_gather(g, src)
        q = _tc_step(h0, q, gsrc, W_m)

    parts = _sc_scatter(q, dst, zeros)
    g = _tc_add(parts)
    gsrc = _sc_gather(g, src)
    ef6 = _tc_ef(h0, q, gsrc)

    parts = _sc_scatter(ef6, dst, zeros)
    return _tc_final(node_feats, parts, W_a[:D], W_a[D:])
